# baseline (speedup 1.0000x reference)
import jax
import jax.numpy as jnp
from jax.experimental import pallas as pl
from jax.experimental.pallas import tpu as pltpu

_C = 256
_PIX = 256

_OFFS = tuple((ky - 1) * 16 + (kx - 1) for ky in range(3) for kx in range(3))


def _row_mask(rows):
    r = jax.lax.broadcasted_iota(jnp.int32, (rows, 1), 0)
    w = r & 15
    h = (r >> 4) & 15
    return (w < 14) & (h < 14)


def _shift(x, d):
    if d == 0:
        return x
    return pltpu.roll(x, (-d) % x.shape[0], 0)


def _conv3x3(x, taps_ref, b, mask):
    acc = jnp.dot(_shift(x, _OFFS[0]), taps_ref[0],
                  preferred_element_type=jnp.float32)
    for t in range(1, 9):
        acc = acc + jnp.dot(_shift(x, _OFFS[t]), taps_ref[t],
                            preferred_element_type=jnp.float32)
    return jnp.where(mask, jnp.maximum(acc + b, 0.0), 0.0)


def _body(x_ref, t1, b1r, t2, b2r, t3, b3r, t4, b4r, tt, btr, w5r, b5r,
          o_ee, o_eo, o_oe, o_oo):
    bt = x_ref.shape[0]
    rows = bt * _PIX
    mask = _row_mask(rows)
    x = x_ref[...].reshape(rows, _C)
    h = _conv3x3(x, t1, b1r[...], mask)
    h = _conv3x3(h, t2, b2r[...], mask)
    h = _conv3x3(h, t3, b3r[...], mask)
    h = _conv3x3(h, t4, b4r[...], mask)

    def tap(t, d):
        return jnp.dot(_shift(h, d), tt[t], preferred_element_type=jnp.float32)

    bt_vec = btr[...]
    ee = tap(4, 0)
    eo = tap(3, 0) + tap(5, 1)
    oe = tap(1, 0) + tap(7, 16)
    oo = tap(0, 0) + tap(2, 1) + tap(6, 16) + tap(8, 17)
    w5 = w5r[...]
    b5 = b5r[...]
    for p, oref in ((ee, o_ee), (eo, o_eo), (oe, o_oe), (oo, o_oo)):
        p = jnp.maximum(p + bt_vec, 0.0)
        s = jax.nn.sigmoid(jnp.dot(p, w5, preferred_element_type=jnp.float32)
                           + b5)
        oref[...] = s.reshape(bt, _PIX, 8)


def _conv_taps(w):
    return jnp.transpose(w, (2, 3, 1, 0)).reshape(9, _C, _C)


def kernel(features, W1, b1, W2, b2, W3, b3, W4, b4, Wt, bt, W5, b5):
    n = features.shape[0]
    x = jnp.transpose(features, (0, 2, 3, 1))
    x = jnp.pad(x, ((0, 0), (0, 2), (0, 2), (0, 0)))
    x = x.reshape(n, _PIX, _C)

    t1, t2, t3, t4 = map(_conv_taps, (W1, W2, W3, W4))
    ttaps = jnp.flip(Wt, axis=(2, 3)).transpose(2, 3, 0, 1).reshape(9, _C, _C)
    w5 = jnp.pad(W5[:, :, 0, 0].T, ((0, 0), (0, 5)))
    b5p = jnp.pad(b5, (0, 5)).reshape(1, 8)
    biases = [b.reshape(1, _C) for b in (b1, b2, b3, b4, bt)]

    bt_sz = 8 if n % 8 == 0 else (4 if n % 4 == 0 else (2 if n % 2 == 0 else 1))
    grid = (n // bt_sz,)

    full = lambda *shape: pl.BlockSpec(shape, lambda i: (0,) * len(shape))
    batched = lambda *shape: pl.BlockSpec((bt_sz,) + shape,
                                          lambda i: (i,) + (0,) * len(shape))
    out_sds = jax.ShapeDtypeStruct((n, _PIX, 8), jnp.float32)
    outs = pl.pallas_call(
        _body,
        grid=grid,
        in_specs=[
            batched(_PIX, _C),
            full(9, _C, _C), full(1, _C),
            full(9, _C, _C), full(1, _C),
            full(9, _C, _C), full(1, _C),
            full(9, _C, _C), full(1, _C),
            full(9, _C, _C), full(1, _C),
            full(_C, 8), full(1, 8),
        ],
        out_specs=[batched(_PIX, 8)] * 4,
        out_shape=[out_sds] * 4,
        compiler_params=pltpu.CompilerParams(
            dimension_semantics=("parallel",)),
    )(x, t1, biases[0], t2, biases[1], t3, biases[2], t4, biases[3],
      ttaps, biases[4], w5, b5p)

    sub = [o[:, :, :3].reshape(n, 16, 16, 3)[:, :14, :14, :]
           .transpose(0, 3, 1, 2) for o in outs]
    ee, eo, oe, oo = sub
    even = jnp.stack([ee, eo], axis=-1).reshape(n, 3, 14, 28)
    odd = jnp.stack([oe, oo], axis=-1).reshape(n, 3, 14, 28)
    return jnp.stack([even, odd], axis=3).reshape(n, 3, 28, 28)


# speedup vs baseline: 1.0884x; 1.0884x over previous
import jax
import jax.numpy as jnp
from jax import lax
from jax.experimental import pallas as pl
from jax.experimental.pallas import tpu as pltpu

_C = 256
_PIX = 256

_OFFS = tuple((ky - 1) * 16 + (kx - 1) for ky in range(3) for kx in range(3))

_TDIMS = (((0,), (0,)), ((), ()))


def _row_mask(rows):
    r = jax.lax.broadcasted_iota(jnp.int32, (rows, 1), 0)
    w = r & 15
    h = (r >> 4) & 15
    return ((w < 14) & (h < 14)).astype(jnp.bfloat16)


def _roll(x, d):
    if d % x.shape[0] == 0:
        return x
    return pltpu.roll(x, (-d) % x.shape[0], 0)


def _conv3x3(x, taps_ref, b, mask):
    acc = None
    for dj in (-1, 0, 1):
        xj = _roll(x, dj)
        for di in (-1, 0, 1):
            t = (di + 1) * 3 + (dj + 1)
            y = jnp.dot(_roll(xj, 16 * di), taps_ref[t],
                        preferred_element_type=jnp.float32)
            acc = y if acc is None else acc + y
    return (jnp.maximum(acc + b, 0.0).astype(jnp.bfloat16)) * mask


def _body(x_ref, s_ref, eye_ref, t1, b1r, t2, b2r, t3, b3r, t4, b4r,
          tt, btr, w5r, b5r, o_ee, o_eo, o_oe, o_oo):
    bt = x_ref.shape[0]
    rows = bt * _PIX
    mask = _row_mask(rows)

    xr = x_ref[...].reshape(bt * _C, 196).astype(jnp.bfloat16)
    tmp = jnp.dot(xr, s_ref[...], preferred_element_type=jnp.float32) \
        .astype(jnp.bfloat16).reshape(bt, _C, _PIX)
    eye = eye_ref[...]
    x = jnp.concatenate(
        [lax.dot_general(tmp[b], eye, _TDIMS,
                         preferred_element_type=jnp.float32)
         .astype(jnp.bfloat16)
         for b in range(bt)], axis=0)

    h = _conv3x3(x, t1, b1r[...], mask)
    h = _conv3x3(h, t2, b2r[...], mask)
    h = _conv3x3(h, t3, b3r[...], mask)
    h = _conv3x3(h, t4, b4r[...], mask)

    def tap(t, d):
        return jnp.dot(_roll(h, d), tt[t], preferred_element_type=jnp.float32)

    bt_vec = btr[...]
    ee = tap(4, 0)
    eo = tap(3, 0) + tap(5, 1)
    oe = tap(1, 0) + tap(7, 16)
    oo = tap(0, 0) + tap(2, 1) + tap(6, 16) + tap(8, 17)
    w5 = w5r[...]
    b5 = b5r[...]
    for p, oref in ((ee, o_ee), (eo, o_eo), (oe, o_oe), (oo, o_oo)):
        p = jnp.maximum(p + bt_vec, 0.0).astype(jnp.bfloat16)
        s = jax.nn.sigmoid(jnp.dot(p, w5, preferred_element_type=jnp.float32)
                           + b5)
        oref[...] = s.reshape(bt, _PIX, 8)


def _conv_taps(w):
    return jnp.transpose(w, (2, 3, 1, 0)).reshape(9, _C, _C).astype(jnp.bfloat16)


def kernel(features, W1, b1, W2, b2, W3, b3, W4, b4, Wt, bt, W5, b5):
    n = features.shape[0]
    x = features.reshape(n, _C, 196)

    t1, t2, t3, t4 = map(_conv_taps, (W1, W2, W3, W4))
    ttaps = jnp.flip(Wt, axis=(2, 3)).transpose(2, 3, 0, 1) \
        .reshape(9, _C, _C).astype(jnp.bfloat16)
    w5 = jnp.pad(W5[:, :, 0, 0].T, ((0, 0), (0, 5))).astype(jnp.bfloat16)
    b5p = jnp.pad(b5, (0, 5)).reshape(1, 8)
    biases = [b.reshape(1, _C) for b in (b1, b2, b3, b4, bt)]

    p = jnp.arange(196)
    g = (p // 14) * 16 + (p % 14)
    smat = jnp.zeros((196, _PIX), jnp.bfloat16).at[p, g].set(1)
    eye = jnp.eye(_C, dtype=jnp.bfloat16)

    bt_sz = 8 if n % 8 == 0 else (4 if n % 4 == 0 else (2 if n % 2 == 0 else 1))
    grid = (n // bt_sz,)

    full = lambda *shape: pl.BlockSpec(shape, lambda i: (0,) * len(shape))
    batched = lambda *shape: pl.BlockSpec((bt_sz,) + shape,
                                          lambda i: (i,) + (0,) * len(shape))
    out_sds = jax.ShapeDtypeStruct((n, _PIX, 8), jnp.float32)
    outs = pl.pallas_call(
        _body,
        grid=grid,
        in_specs=[
            batched(_C, 196),
            full(196, _PIX), full(_C, _C),
            full(9, _C, _C), full(1, _C),
            full(9, _C, _C), full(1, _C),
            full(9, _C, _C), full(1, _C),
            full(9, _C, _C), full(1, _C),
            full(9, _C, _C), full(1, _C),
            full(_C, 8), full(1, 8),
        ],
        out_specs=[batched(_PIX, 8)] * 4,
        out_shape=[out_sds] * 4,
        compiler_params=pltpu.CompilerParams(
            dimension_semantics=("parallel",)),
    )(x, smat, eye, t1, biases[0], t2, biases[1], t3, biases[2],
      t4, biases[3], ttaps, biases[4], w5, b5p)

    sub = [o[:, :, :3].reshape(n, 16, 16, 3)[:, :14, :14, :]
           .transpose(0, 3, 1, 2) for o in outs]
    ee, eo, oe, oo = sub
    even = jnp.stack([ee, eo], axis=-1).reshape(n, 3, 14, 28)
    odd = jnp.stack([oe, oo], axis=-1).reshape(n, 3, 14, 28)
    return jnp.stack([even, odd], axis=3).reshape(n, 3, 28, 28)


# speedup vs baseline: 1.1642x; 1.0696x over previous
import jax
import jax.numpy as jnp
from jax import lax
from jax.experimental import pallas as pl
from jax.experimental.pallas import tpu as pltpu

_C = 256
_PIX = 256

_OFFS = tuple((ky - 1) * 16 + (kx - 1) for ky in range(3) for kx in range(3))

_TDIMS = (((0,), (0,)), ((), ()))


def _row_mask(rows):
    r = jax.lax.broadcasted_iota(jnp.int32, (rows, 1), 0)
    w = r & 15
    h = (r >> 4) & 15
    return ((w < 14) & (h < 14)).astype(jnp.bfloat16)


def _roll(x, d):
    if d % x.shape[0] == 0:
        return x
    return pltpu.roll(x, (-d) % x.shape[0], 0)


def _halo(x):
    hp = jnp.pad(x, ((16, 16), (0, 0)))
    return {dj: _roll(hp, dj) for dj in (-1, 0, 1)}


def _tap_operand(hs, di, dj, rows):
    return lax.slice_in_dim(hs[dj], 16 + 16 * di, 16 + 16 * di + rows, axis=0)


def _conv3x3(x, taps_ref, b, mask):
    rows = x.shape[0]
    hs = _halo(x)
    acc = None
    for di in (-1, 0, 1):
        for dj in (-1, 0, 1):
            t = (di + 1) * 3 + (dj + 1)
            y = jnp.dot(_tap_operand(hs, di, dj, rows), taps_ref[t],
                        preferred_element_type=jnp.float32)
            acc = y if acc is None else acc + y
    return (jnp.maximum(acc + b, 0.0).astype(jnp.bfloat16)) * mask


def _body(x_ref, s_ref, eye_ref, t1, b1r, t2, b2r, t3, b3r, t4, b4r,
          tt, btr, w5r, b5r, o_ee, o_eo, o_oe, o_oo):
    bt = x_ref.shape[0]
    rows = bt * _PIX
    mask = _row_mask(rows)

    xr = x_ref[...].reshape(bt * _C, 196).astype(jnp.bfloat16)
    tmp = jnp.dot(xr, s_ref[...], preferred_element_type=jnp.float32) \
        .astype(jnp.bfloat16).reshape(bt, _C, _PIX)
    eye = eye_ref[...]
    x = jnp.concatenate(
        [lax.dot_general(tmp[b], eye, _TDIMS,
                         preferred_element_type=jnp.float32)
         .astype(jnp.bfloat16)
         for b in range(bt)], axis=0)

    h = _conv3x3(x, t1, b1r[...], mask)
    h = _conv3x3(h, t2, b2r[...], mask)
    h = _conv3x3(h, t3, b3r[...], mask)
    h = _conv3x3(h, t4, b4r[...], mask)

    hs = _halo(h)

    def tap(t, di, dj):
        return jnp.dot(_tap_operand(hs, di, dj, rows), tt[t],
                       preferred_element_type=jnp.float32)

    bt_vec = btr[...]
    ee = tap(4, 0, 0)
    eo = tap(3, 0, 0) + tap(5, 0, 1)
    oe = tap(1, 0, 0) + tap(7, 1, 0)
    oo = tap(0, 0, 0) + tap(2, 0, 1) + tap(6, 1, 0) + tap(8, 1, 1)
    w5 = w5r[...]
    b5 = b5r[...]
    for p, oref in ((ee, o_ee), (eo, o_eo), (oe, o_oe), (oo, o_oo)):
        p = jnp.maximum(p + bt_vec, 0.0).astype(jnp.bfloat16)
        s = jax.nn.sigmoid(jnp.dot(p, w5, preferred_element_type=jnp.float32)
                           + b5)
        oref[...] = s.reshape(bt, _PIX, 8)


def _conv_taps(w):
    return jnp.transpose(w, (2, 3, 1, 0)).reshape(9, _C, _C).astype(jnp.bfloat16)


def kernel(features, W1, b1, W2, b2, W3, b3, W4, b4, Wt, bt, W5, b5):
    n = features.shape[0]
    x = features.reshape(n, _C, 196)

    t1, t2, t3, t4 = map(_conv_taps, (W1, W2, W3, W4))
    ttaps = jnp.flip(Wt, axis=(2, 3)).transpose(2, 3, 0, 1) \
        .reshape(9, _C, _C).astype(jnp.bfloat16)
    w5 = jnp.pad(W5[:, :, 0, 0].T, ((0, 0), (0, 5))).astype(jnp.bfloat16)
    b5p = jnp.pad(b5, (0, 5)).reshape(1, 8)
    biases = [b.reshape(1, _C) for b in (b1, b2, b3, b4, bt)]

    p = jnp.arange(196)
    g = (p // 14) * 16 + (p % 14)
    smat = jnp.zeros((196, _PIX), jnp.bfloat16).at[p, g].set(1)
    eye = jnp.eye(_C, dtype=jnp.bfloat16)

    bt_sz = 8 if n % 8 == 0 else (4 if n % 4 == 0 else (2 if n % 2 == 0 else 1))
    grid = (n // bt_sz,)

    full = lambda *shape: pl.BlockSpec(shape, lambda i: (0,) * len(shape))
    batched = lambda *shape: pl.BlockSpec((bt_sz,) + shape,
                                          lambda i: (i,) + (0,) * len(shape))
    out_sds = jax.ShapeDtypeStruct((n, _PIX, 8), jnp.float32)
    outs = pl.pallas_call(
        _body,
        grid=grid,
        in_specs=[
            batched(_C, 196),
            full(196, _PIX), full(_C, _C),
            full(9, _C, _C), full(1, _C),
            full(9, _C, _C), full(1, _C),
            full(9, _C, _C), full(1, _C),
            full(9, _C, _C), full(1, _C),
            full(9, _C, _C), full(1, _C),
            full(_C, 8), full(1, 8),
        ],
        out_specs=[batched(_PIX, 8)] * 4,
        out_shape=[out_sds] * 4,
        compiler_params=pltpu.CompilerParams(
            dimension_semantics=("parallel",)),
    )(x, smat, eye, t1, biases[0], t2, biases[1], t3, biases[2],
      t4, biases[3], ttaps, biases[4], w5, b5p)

    sub = [o[:, :, :3].reshape(n, 16, 16, 3)[:, :14, :14, :]
           .transpose(0, 3, 1, 2) for o in outs]
    ee, eo, oe, oo = sub
    even = jnp.stack([ee, eo], axis=-1).reshape(n, 3, 14, 28)
    odd = jnp.stack([oe, oo], axis=-1).reshape(n, 3, 14, 28)
    return jnp.stack([even, odd], axis=3).reshape(n, 3, 28, 28)
